# speedup vs baseline: 1.9438x; 1.9438x over previous
import functools

import jax
import jax.numpy as jnp
from jax import lax
from jax.experimental import pallas as pl
from jax.experimental.pallas import tpu as pltpu
from jax.experimental.pallas import tpu_sc as plsc

Q = 8
KROWS = 1025
H = 1024
B = 16
S = 2048
L = 16

NC = 2
NS = 16
NW = NC * NS

ROWS_PER_W = (B * S) // NW
C = 8
N_CHUNKS = ROWS_PER_W // C

_mesh = plsc.VectorSubcoreMesh(core_axis_name="c", subcore_axis_name="s")


@functools.partial(
    pl.kernel,
    out_type=jax.ShapeDtypeStruct((B, S, H), jnp.float32),
    mesh=_mesh,
    scratch_types=[
        pltpu.VMEM((Q, ROWS_PER_W), jnp.int32),
        pltpu.VMEM((Q, C, H), jnp.float32),
        pltpu.VMEM((C, H), jnp.float32),
        pltpu.SemaphoreType.DMA,
    ],
)
def _embed_sum(x_hbm, tables_hbm, out_hbm, idx_v, bufs_v, obuf_v, sem):
    wid = lax.axis_index("s") * NC + lax.axis_index("c")
    b = wid // 2
    s0 = (wid % 2) * ROWS_PER_W

    pltpu.sync_copy(x_hbm.at[b, :, pl.ds(s0, ROWS_PER_W)], idx_v)
    for q in range(1, Q):
        def add_off(g, carry, q=q):
            o = g * L
            idx_v[q, pl.ds(o, L)] = idx_v[q, pl.ds(o, L)] + (q * KROWS)
            return carry
        lax.fori_loop(0, ROWS_PER_W // L, add_off, 0)

    def chunk(i, carry):
        cbase = i * C
        copies = []
        for q in range(Q):
            cp = pltpu.make_async_copy(
                tables_hbm.at[idx_v.at[q, pl.ds(cbase, C)]],
                bufs_v.at[q],
                sem,
            )
            cp.start()
            copies.append(cp)
        for cp in copies:
            cp.wait()

        for c in range(C):
            def sum_row(g, carry2, c=c):
                o = g * L
                v = bufs_v[0, c, pl.ds(o, L)]
                for q in range(1, Q):
                    v = v + bufs_v[q, c, pl.ds(o, L)]
                obuf_v[c, pl.ds(o, L)] = v
                return carry2
            lax.fori_loop(0, H // L, sum_row, 0)

        pltpu.sync_copy(obuf_v, out_hbm.at[b, pl.ds(s0 + cbase, C)])
        return carry

    lax.fori_loop(0, N_CHUNKS, chunk, 0)


def kernel(x, tables):
    embds = _embed_sum(x, tables.reshape(Q * KROWS, H))
    return (x, embds)


# speedup vs baseline: 3.3781x; 1.7379x over previous
import functools

import jax
import jax.numpy as jnp
from jax import lax
from jax.experimental import pallas as pl
from jax.experimental.pallas import tpu as pltpu
from jax.experimental.pallas import tpu_sc as plsc

Q = 8
KROWS = 1025
H = 1024
B = 16
S = 2048
L = 16

NC = 2
NS = 16
NW = NC * NS

ROWS_PER_W = (B * S) // NW
C = 4
QC = Q * C
N_CHUNKS = ROWS_PER_W // C

_mesh = plsc.VectorSubcoreMesh(core_axis_name="c", subcore_axis_name="s")


@functools.partial(
    pl.kernel,
    out_type=jax.ShapeDtypeStruct((B, S, H), jnp.float32),
    mesh=_mesh,
    scratch_types=[
        pltpu.VMEM((Q, ROWS_PER_W), jnp.int32),
        pltpu.VMEM((Q, C, H), jnp.float32),
        pltpu.VMEM((Q, C, H), jnp.float32),
        pltpu.VMEM((C, H), jnp.float32),
        pltpu.VMEM((C, H), jnp.float32),
        pltpu.SemaphoreType.DMA,
        pltpu.SemaphoreType.DMA,
        pltpu.SemaphoreType.DMA,
        pltpu.SemaphoreType.DMA,
    ],
)
def _embed_sum(x_hbm, tab_hbm, out_hbm, idx_all, bufA, bufB, oA, oB,
               sgA, sgB, soA, soB):
    wid = lax.axis_index("s") * NC + lax.axis_index("c")
    b = wid // 2
    s0 = (wid % 2) * ROWS_PER_W

    pltpu.sync_copy(x_hbm.at[b, :, pl.ds(s0, ROWS_PER_W)], idx_all)
    for q in range(1, Q):
        def add_off(g, carry, q=q):
            o = g * L
            idx_all[q, pl.ds(o, L)] = idx_all[q, pl.ds(o, L)] + (q * KROWS)
            return carry
        lax.fori_loop(0, ROWS_PER_W // L, add_off, 0)

    def g_start(i, buf, sem):
        for q in range(Q):
            pltpu.make_async_copy(
                tab_hbm.at[idx_all.at[q, pl.ds(i * C, C)]],
                buf.at[q], sem).start()

    def g_wait(buf, sem):
        for q in range(Q):
            pltpu.make_async_copy(
                tab_hbm.at[idx_all.at[q, pl.ds(0, C)]],
                buf.at[q], sem).wait()

    def o_start(i, o, sem):
        pltpu.make_async_copy(o, out_hbm.at[b, pl.ds(s0 + i * C, C)],
                              sem).start()

    def o_wait(o, sem):
        pltpu.make_async_copy(o, out_hbm.at[b, pl.ds(s0, C)], sem).wait()

    def do_sum(buf, o):
        for c in range(C):
            def srow(g, carry, c=c):
                p = pl.ds(g * L, L)
                v = buf[0, c, p]
                for q in range(1, Q):
                    v = v + buf[q, c, p]
                o[c, p] = v
                return carry
            lax.fori_loop(0, H // L, srow, 0)

    g_start(0, bufA, sgA)
    g_start(1, bufB, sgB)

    def body(j, carry):
        ca = 2 * j
        g_wait(bufA, sgA)

        @pl.when(j > 0)
        def _():
            o_wait(oA, soA)

        do_sum(bufA, oA)
        o_start(ca, oA, soA)

        @pl.when(j < N_CHUNKS // 2 - 1)
        def _():
            g_start(ca + 2, bufA, sgA)

        g_wait(bufB, sgB)

        @pl.when(j > 0)
        def _():
            o_wait(oB, soB)

        do_sum(bufB, oB)
        o_start(ca + 1, oB, soB)

        @pl.when(j < N_CHUNKS // 2 - 1)
        def _():
            g_start(ca + 3, bufB, sgB)

        return carry

    lax.fori_loop(0, N_CHUNKS // 2, body, 0)
    o_wait(oA, soA)
    o_wait(oB, soB)


def kernel(x, tables):
    embds = _embed_sum(x, tables.reshape(Q * KROWS, H))
    return (x, embds)
